# Initial kernel scaffold; baseline (speedup 1.0000x reference)
import jax
import jax.numpy as jnp
from jax import lax
from jax.experimental import pallas as pl
from jax.experimental.pallas import tpu as pltpu
from jax.experimental.pallas import tpu_sc as plsc

_INPUT_DIM = 3
_NUM_LEVELS = 16
_LEVEL_DIM = 2
_BASE_RES = 16
_HASH_MASK = (1 << 19) - 1
_P1 = jnp.int32(jnp.uint32(2654435761).astype(jnp.int32))
_P2 = jnp.int32(805459861)

_OFFS = [0, 4920, 40864, 315496, 839784, 1364072, 1888360, 2412648,
         2936936, 3461224, 3985512, 4509800, 5034088, 5558376, 6082664,
         6606952, 7131240]
_TOTAL_ROWS = _OFFS[-1]
_DENSE_LEVELS = 3

_NC, _NS, _L = 2, 16, 16
_NW = _NC * _NS
_NCORN = 8
_IDXROW = _NCORN * _L


def _body(x_hbm, emb_hbm, o_hbm, pts_v, idx_v, w_v, rows_v, out_v, gsem):
    chunk = x_hbm.shape[1] // _NW
    ngroups = chunk // _L
    wid = lax.axis_index("s") * _NC + lax.axis_index("c")
    base = wid * chunk
    pltpu.sync_copy(x_hbm.at[:, pl.ds(base, chunk)], pts_v)
    iota = lax.iota(jnp.int32, _L)

    @pl.loop(0, ngroups)
    def _group(g):
        gb = g * _L
        xs = pts_v[0, pl.ds(gb, _L)]
        ys = pts_v[1, pl.ds(gb, _L)]
        zs = pts_v[2, pl.ds(gb, _L)]
        x0 = (xs + 1.0) * 0.5
        y0 = (ys + 1.0) * 0.5
        z0 = (zs + 1.0) * 0.5

        for l in range(_NUM_LEVELS):
            scale = float(2.0 ** l * _BASE_RES - 1.0)
            px = x0 * scale + 0.5
            py = y0 * scale + 0.5
            pz = z0 * scale + 0.5
            ix = px.astype(jnp.int32)
            iy = py.astype(jnp.int32)
            iz = pz.astype(jnp.int32)
            fx = px - ix.astype(jnp.float32)
            fy = py - iy.astype(jnp.float32)
            fz = pz - iz.astype(jnp.float32)
            if l < _DENSE_LEVELS:
                mult = 2 ** l * _BASE_RES + 1
                aa = (ix + _OFFS[l], ix + (_OFFS[l] + 1))
                b0 = iy * mult
                c0 = iz * (mult * mult)
                bb = (b0, b0 + mult)
                cc = (c0, c0 + mult * mult)
            else:
                aa = (ix, ix + 1)
                b0 = iy * _P1
                c0 = iz * _P2
                bb = (b0, b0 + _P1)
                cc = (c0, c0 + _P2)
            wx = (1.0 - fx, fx)
            wy = (1.0 - fy, fy)
            wz = (1.0 - fz, fz)
            wxy = [wx[cx] * wy[cy] for cy in range(2) for cx in range(2)]
            for c in range(_NCORN):
                cx, cy, cz = c & 1, (c >> 1) & 1, (c >> 2) & 1
                if l < _DENSE_LEVELS:
                    idx = aa[cx] + bb[cy] + cc[cz]
                else:
                    idx = ((aa[cx] ^ bb[cy] ^ cc[cz]) & _HASH_MASK) + _OFFS[l]
                idx_v[l, pl.ds(c * _L, _L)] = idx
                w_v[pl.ds((l * _NCORN + c) * _L, _L)] = wxy[cy * 2 + cx] * wz[cz]

        cps = [
            pltpu.async_copy(
                emb_hbm.at[idx_v.at[l]],
                rows_v.at[pl.ds(l * _IDXROW, _IDXROW), :],
                gsem,
            )
            for l in range(_NUM_LEVELS)
        ]
        for cp in cps:
            cp.wait()

        chvecs = [jnp.full((_L,), ch, jnp.int32) for ch in range(6)]
        for l in range(_NUM_LEVELS):
            accs = [jnp.zeros((_L,), jnp.float32) for _ in range(6)]
            for c in range(_NCORN):
                w = w_v[pl.ds((l * _NCORN + c) * _L, _L)]
                rvec = iota + (l * _IDXROW + c * _L)
                for ch in range(6):
                    v = plsc.load_gather(rows_v, [rvec, chvecs[ch]])
                    accs[ch] = accs[ch] + w * v
            for ch in range(6):
                j = (2 * l + (ch % 2)) * 3 + ch // 2
                plsc.store_scatter(out_v, [iota, jnp.full((_L,), j, jnp.int32)],
                                   accs[ch])

        pltpu.sync_copy(out_v, o_hbm.at[pl.ds(base + gb, _L), :])


@jax.jit
def _encode(x_t, emb6):
    b = x_t.shape[1]
    f = _NUM_LEVELS * _LEVEL_DIM * _INPUT_DIM
    run = pl.kernel(
        _body,
        out_type=jax.ShapeDtypeStruct((b, f), jnp.float32),
        mesh=plsc.VectorSubcoreMesh(core_axis_name="c", subcore_axis_name="s"),
        scratch_types=[
            pltpu.VMEM((_INPUT_DIM, b // _NW), jnp.float32),
            pltpu.VMEM((_NUM_LEVELS, _IDXROW), jnp.int32),
            pltpu.VMEM((_NUM_LEVELS * _IDXROW,), jnp.float32),
            pltpu.VMEM((_NUM_LEVELS * _IDXROW, 6), jnp.float32),
            pltpu.VMEM((_L, f), jnp.float32),
            pltpu.SemaphoreType.DMA,
        ],
    )
    return run(x_t, emb6)


def kernel(inputs, emb_x, emb_y, emb_z):
    b = inputs.shape[0]
    x_t = inputs.T
    emb6 = jnp.concatenate([emb_x, emb_y, emb_z], axis=1)
    out = _encode(x_t, emb6)
    return out.reshape(b, _NUM_LEVELS * _LEVEL_DIM, _INPUT_DIM)


# speedup vs baseline: 1.9405x; 1.9405x over previous
import dataclasses

import jax
import jax.numpy as jnp
from jax import lax
from jax.experimental import pallas as pl
from jax.experimental.pallas import tpu as pltpu
from jax.experimental.pallas import tpu_sc as plsc

_INPUT_DIM = 3
_NUM_LEVELS = 16
_LEVEL_DIM = 2
_BASE_RES = 16
_HASH_MASK = (1 << 19) - 1
_P1 = 2654435761 - (1 << 32)
_P2 = 805459861

_OFFS = [0, 4920, 40864, 315496, 839784, 1364072, 1888360, 2412648,
         2936936, 3461224, 3985512, 4509800, 5034088, 5558376, 6082664,
         6606952, 7131240]
_TOTAL_ROWS = _OFFS[-1]
_DENSE_LEVELS = 3

_NC, _NS, _L = 2, 16, 16
_NW = _NC * _NS
_NCORN = 8
_IDXROW = _NCORN * _L


def _body(x_hbm, emb_hbm, o_hbm, pts_v, idx_v, w_v, rows_v, out_v, gsem):
    chunk = x_hbm.shape[1] // _NW
    ngroups = chunk // _L
    wid = lax.axis_index("s") * _NC + lax.axis_index("c")
    base = wid * chunk
    pltpu.sync_copy(x_hbm.at[:, pl.ds(base, chunk)], pts_v)
    iota = lax.iota(jnp.int32, _L)

    @pl.loop(0, ngroups)
    def _group(g):
        gb = g * _L
        xs = pts_v[0, pl.ds(gb, _L)]
        ys = pts_v[1, pl.ds(gb, _L)]
        zs = pts_v[2, pl.ds(gb, _L)]
        x0 = (xs + 1.0) * 0.5
        y0 = (ys + 1.0) * 0.5
        z0 = (zs + 1.0) * 0.5

        for l in range(_NUM_LEVELS):
            scale = float(2.0 ** l * _BASE_RES - 1.0)
            px = x0 * scale + 0.5
            py = y0 * scale + 0.5
            pz = z0 * scale + 0.5
            ix = px.astype(jnp.int32)
            iy = py.astype(jnp.int32)
            iz = pz.astype(jnp.int32)
            fx = px - ix.astype(jnp.float32)
            fy = py - iy.astype(jnp.float32)
            fz = pz - iz.astype(jnp.float32)
            if l < _DENSE_LEVELS:
                mult = 2 ** l * _BASE_RES + 1
                aa = (ix + _OFFS[l], ix + (_OFFS[l] + 1))
                b0 = iy * mult
                c0 = iz * (mult * mult)
                bb = (b0, b0 + mult)
                cc = (c0, c0 + mult * mult)
            else:
                aa = (ix, ix + 1)
                b0 = iy * _P1
                c0 = iz * _P2
                bb = (b0, b0 + _P1)
                cc = (c0, c0 + _P2)
            wx = (1.0 - fx, fx)
            wy = (1.0 - fy, fy)
            wz = (1.0 - fz, fz)
            wxy = [wx[cx] * wy[cy] for cy in range(2) for cx in range(2)]
            for c in range(_NCORN):
                cx, cy, cz = c & 1, (c >> 1) & 1, (c >> 2) & 1
                if l < _DENSE_LEVELS:
                    idx = aa[cx] + bb[cy] + cc[cz]
                else:
                    idx = ((aa[cx] ^ bb[cy] ^ cc[cz]) & _HASH_MASK) + _OFFS[l]
                idx_v[l, pl.ds(c * _L, _L)] = idx
                w_v[pl.ds((l * _NCORN + c) * _L, _L)] = wxy[cy * 2 + cx] * wz[cz]

        cps = [
            pltpu.async_copy(
                emb_hbm.at[idx_v.at[l]],
                rows_v.at[pl.ds(l * _IDXROW, _IDXROW), :],
                gsem,
            )
            for l in range(_NUM_LEVELS)
        ]
        for cp in cps:
            cp.wait()

        chvecs = [jnp.full((_L,), ch, jnp.int32) for ch in range(6)]
        for l in range(_NUM_LEVELS):
            accs = [jnp.zeros((_L,), jnp.float32) for _ in range(6)]
            for c in range(_NCORN):
                w = w_v[pl.ds((l * _NCORN + c) * _L, _L)]
                rvec = iota + (l * _IDXROW + c * _L)
                for ch in range(6):
                    v = plsc.load_gather(rows_v, [rvec, chvecs[ch]])
                    accs[ch] = accs[ch] + w * v
            for ch in range(6):
                j = (2 * l + (ch % 2)) * 3 + ch // 2
                plsc.store_scatter(out_v, [iota, jnp.full((_L,), j, jnp.int32)],
                                   accs[ch])

        pltpu.sync_copy(out_v, o_hbm.at[pl.ds(base + gb, _L), :])


@jax.jit
def _encode(x_t, emb6):
    b = x_t.shape[1]
    f = _NUM_LEVELS * _LEVEL_DIM * _INPUT_DIM
    cp = pltpu.CompilerParams()
    if "needs_layout_passes" in pltpu.CompilerParams.__dataclass_fields__:
        cp = dataclasses.replace(cp, needs_layout_passes=False)
    if "use_tc_tiling_on_sc" in pltpu.CompilerParams.__dataclass_fields__:
        cp = dataclasses.replace(cp, use_tc_tiling_on_sc=False)
    run = pl.kernel(
        _body,
        out_type=jax.ShapeDtypeStruct((b, f), jnp.float32),
        compiler_params=cp,
        mesh=plsc.VectorSubcoreMesh(core_axis_name="c", subcore_axis_name="s"),
        scratch_types=[
            pltpu.VMEM((_INPUT_DIM, b // _NW), jnp.float32),
            pltpu.VMEM((_NUM_LEVELS, _IDXROW), jnp.int32),
            pltpu.VMEM((_NUM_LEVELS * _IDXROW,), jnp.float32),
            pltpu.VMEM((_NUM_LEVELS * _IDXROW, 6), jnp.float32),
            pltpu.VMEM((_L, f), jnp.float32),
            pltpu.SemaphoreType.DMA,
        ],
    )
    return run(x_t, emb6)


def kernel(inputs, emb_x, emb_y, emb_z):
    b = inputs.shape[0]
    x_t = inputs.T
    emb6 = jnp.concatenate([emb_x, emb_y, emb_z], axis=1)
    out = _encode(x_t, emb6)
    return out.reshape(b, _NUM_LEVELS * _LEVEL_DIM, _INPUT_DIM)


# speedup vs baseline: 3.3154x; 1.7085x over previous
import dataclasses

import jax
import jax.numpy as jnp
from jax import lax
from jax.experimental import pallas as pl
from jax.experimental.pallas import tpu as pltpu
from jax.experimental.pallas import tpu_sc as plsc

_INPUT_DIM = 3
_NUM_LEVELS = 16
_LEVEL_DIM = 2
_BASE_RES = 16
_HASH_MASK = (1 << 19) - 1
_P1 = 2654435761 - (1 << 32)
_P2 = 805459861

_OFFS = [0, 4920, 40864, 315496, 839784, 1364072, 1888360, 2412648,
         2936936, 3461224, 3985512, 4509800, 5034088, 5558376, 6082664,
         6606952, 7131240]
_TOTAL_ROWS = _OFFS[-1]
_DENSE_LEVELS = 3

_NC, _NS, _L = 2, 16, 16
_NW = _NC * _NS
_NCORN = 8
_IDXROW = _NCORN * _L


_CROWS = 2048


def _interleave_body(c0, c1, c2, c3, c4, c5, e8_hbm, cin_v, o8_v):
    cols = (c0, c1, c2, c3, c4, c5)
    rows = c0.shape[0]
    nchunk = -(-rows // _CROWS)
    tail = rows - (nchunk - 1) * _CROWS
    wid = lax.axis_index("s") * _NC + lax.axis_index("c")
    steps = -(-nchunk // _NW)
    iota = lax.iota(jnp.int32, _L)

    @pl.loop(0, steps)
    def _step(k):
        cid = k * _NW + wid
        rb = cid * _CROWS

        @pl.when(cid < nchunk - 1)
        def _full():
            for ch in range(6):
                pltpu.sync_copy(cols[ch].at[pl.ds(rb, _CROWS)], cin_v.at[ch])
            for vg in range(_CROWS // _L):
                rvec = iota + vg * _L
                for ch in range(6):
                    v = cin_v[ch, pl.ds(vg * _L, _L)]
                    plsc.store_scatter(o8_v, [rvec, jnp.full((_L,), ch, jnp.int32)], v)
            pltpu.sync_copy(o8_v, e8_hbm.at[pl.ds(rb, _CROWS), :])

        @pl.when(cid == nchunk - 1)
        def _tail():
            for ch in range(6):
                pltpu.sync_copy(cols[ch].at[pl.ds(rb, tail)],
                                cin_v.at[ch, pl.ds(0, tail)])
            for vg in range(-(-tail // _L)):
                rvec = iota + vg * _L
                full = (vg + 1) * _L <= tail
                mask = None if full else rvec < tail
                for ch in range(6):
                    v = cin_v[ch, pl.ds(vg * _L, _L)]
                    plsc.store_scatter(o8_v, [rvec, jnp.full((_L,), ch, jnp.int32)],
                                       v, mask=mask)
            pltpu.sync_copy(o8_v.at[pl.ds(0, tail), :],
                            e8_hbm.at[pl.ds(rb, tail), :])


def _body(x_hbm, emb_hbm, o_hbm, pts_v, idx_v, w_v, rows_v, out_v, gsem):
    chunk = x_hbm.shape[1] // _NW
    ngroups = chunk // _L
    wid = lax.axis_index("s") * _NC + lax.axis_index("c")
    base = wid * chunk
    pltpu.sync_copy(x_hbm.at[:, pl.ds(base, chunk)], pts_v)
    iota = lax.iota(jnp.int32, _L)

    @pl.loop(0, ngroups)
    def _group(g):
        gb = g * _L
        xs = pts_v[0, pl.ds(gb, _L)]
        ys = pts_v[1, pl.ds(gb, _L)]
        zs = pts_v[2, pl.ds(gb, _L)]
        x0 = (xs + 1.0) * 0.5
        y0 = (ys + 1.0) * 0.5
        z0 = (zs + 1.0) * 0.5

        for l in range(_NUM_LEVELS):
            scale = float(2.0 ** l * _BASE_RES - 1.0)
            px = x0 * scale + 0.5
            py = y0 * scale + 0.5
            pz = z0 * scale + 0.5
            ix = px.astype(jnp.int32)
            iy = py.astype(jnp.int32)
            iz = pz.astype(jnp.int32)
            fx = px - ix.astype(jnp.float32)
            fy = py - iy.astype(jnp.float32)
            fz = pz - iz.astype(jnp.float32)
            if l < _DENSE_LEVELS:
                mult = 2 ** l * _BASE_RES + 1
                aa = (ix + _OFFS[l], ix + (_OFFS[l] + 1))
                b0 = iy * mult
                c0 = iz * (mult * mult)
                bb = (b0, b0 + mult)
                cc = (c0, c0 + mult * mult)
            else:
                aa = (ix, ix + 1)
                b0 = iy * _P1
                c0 = iz * _P2
                bb = (b0, b0 + _P1)
                cc = (c0, c0 + _P2)
            wx = (1.0 - fx, fx)
            wy = (1.0 - fy, fy)
            wz = (1.0 - fz, fz)
            wxy = [wx[cx] * wy[cy] for cy in range(2) for cx in range(2)]
            for c in range(_NCORN):
                cx, cy, cz = c & 1, (c >> 1) & 1, (c >> 2) & 1
                if l < _DENSE_LEVELS:
                    idx = aa[cx] + bb[cy] + cc[cz]
                else:
                    idx = ((aa[cx] ^ bb[cy] ^ cc[cz]) & _HASH_MASK) + _OFFS[l]
                idx_v[l, pl.ds(c * _L, _L)] = idx
                w_v[pl.ds((l * _NCORN + c) * _L, _L)] = wxy[cy * 2 + cx] * wz[cz]

        cps = [
            pltpu.async_copy(
                emb_hbm.at[idx_v.at[l]],
                rows_v.at[pl.ds(l * _IDXROW, _IDXROW), :],
                gsem,
            )
            for l in range(_NUM_LEVELS)
        ]
        for cp in cps:
            cp.wait()

        chvecs = [jnp.full((_L,), ch, jnp.int32) for ch in range(6)]
        for l in range(_NUM_LEVELS):
            accs = [jnp.zeros((_L,), jnp.float32) for _ in range(6)]
            for c in range(_NCORN):
                w = w_v[pl.ds((l * _NCORN + c) * _L, _L)]
                rvec = iota + (l * _IDXROW + c * _L)
                for ch in range(6):
                    v = plsc.load_gather(rows_v, [rvec, chvecs[ch]])
                    accs[ch] = accs[ch] + w * v
            for ch in range(6):
                j = (2 * l + (ch % 2)) * 3 + ch // 2
                plsc.store_scatter(out_v, [iota, jnp.full((_L,), j, jnp.int32)],
                                   accs[ch])

        pltpu.sync_copy(out_v, o_hbm.at[pl.ds(base + gb, _L), :])


def _sc_params():
    cp = pltpu.CompilerParams()
    if "needs_layout_passes" in pltpu.CompilerParams.__dataclass_fields__:
        cp = dataclasses.replace(cp, needs_layout_passes=False)
    if "use_tc_tiling_on_sc" in pltpu.CompilerParams.__dataclass_fields__:
        cp = dataclasses.replace(cp, use_tc_tiling_on_sc=False)
    return cp


@jax.jit
def _encode(inputs, emb_x, emb_y, emb_z):
    b = inputs.shape[0]
    x_t = inputs.T
    cols = [t[:, d] for t in (emb_x, emb_y, emb_z) for d in range(2)]
    rows = emb_x.shape[0]
    f = _NUM_LEVELS * _LEVEL_DIM * _INPUT_DIM
    mesh = plsc.VectorSubcoreMesh(core_axis_name="c", subcore_axis_name="s")
    interleave = pl.kernel(
        _interleave_body,
        out_type=jax.ShapeDtypeStruct((rows, 8), jnp.float32),
        compiler_params=_sc_params(),
        mesh=mesh,
        scratch_types=[
            pltpu.VMEM((6, _CROWS), jnp.float32),
            pltpu.VMEM((_CROWS, 8), jnp.float32),
        ],
    )
    emb8 = interleave(*cols)
    run = pl.kernel(
        _body,
        out_type=jax.ShapeDtypeStruct((b, f), jnp.float32),
        compiler_params=_sc_params(),
        mesh=mesh,
        scratch_types=[
            pltpu.VMEM((_INPUT_DIM, b // _NW), jnp.float32),
            pltpu.VMEM((_NUM_LEVELS, _IDXROW), jnp.int32),
            pltpu.VMEM((_NUM_LEVELS * _IDXROW,), jnp.float32),
            pltpu.VMEM((_NUM_LEVELS * _IDXROW, 8), jnp.float32),
            pltpu.VMEM((_L, f), jnp.float32),
            pltpu.SemaphoreType.DMA,
        ],
    )
    return run(x_t, emb8)


def kernel(inputs, emb_x, emb_y, emb_z):
    b = inputs.shape[0]
    out = _encode(inputs, emb_x, emb_y, emb_z)
    return out.reshape(b, _NUM_LEVELS * _LEVEL_DIM, _INPUT_DIM)


# speedup vs baseline: 5.2099x; 1.5714x over previous
import dataclasses

import jax
import jax.numpy as jnp
from jax import lax
from jax.experimental import pallas as pl
from jax.experimental.pallas import tpu as pltpu
from jax.experimental.pallas import tpu_sc as plsc

_INPUT_DIM = 3
_NUM_LEVELS = 16
_LEVEL_DIM = 2
_BASE_RES = 16
_HASH_MASK = (1 << 19) - 1
_P1 = 2654435761 - (1 << 32)
_P2 = 805459861

_OFFS = [0, 4920, 40864, 315496, 839784, 1364072, 1888360, 2412648,
         2936936, 3461224, 3985512, 4509800, 5034088, 5558376, 6082664,
         6606952, 7131240]
_TOTAL_ROWS = _OFFS[-1]
_DENSE_LEVELS = 3

_NC, _NS, _L = 2, 16, 16
_NW = _NC * _NS
_NCORN = 8
_IDXROW = _NCORN * _L


_BK = 32768


def _tc_cols_body(xt, yt, zt, o0, o1, o2, o3, o4, o5):
    outs = (o0, o1, o2, o3, o4, o5)
    for i, t in enumerate((xt, yt, zt)):
        outs[2 * i][...] = t[0, :]
        outs[2 * i + 1][...] = t[1, :]


def _split_columns(emb_x, emb_y, emb_z):
    rows = emb_x.shape[0]
    grid = (-(-rows // _BK),)
    in_spec = pl.BlockSpec((2, _BK), lambda j: (0, j))
    out_spec = pl.BlockSpec((_BK,), lambda j: (j,))
    return pl.pallas_call(
        _tc_cols_body,
        grid=grid,
        in_specs=[in_spec] * 3,
        out_specs=[out_spec] * 6,
        out_shape=[jax.ShapeDtypeStruct((rows,), jnp.float32)] * 6,
    )(emb_x.T, emb_y.T, emb_z.T)


_CROWS = 2048


def _interleave_body(c0, c1, c2, c3, c4, c5, e8_hbm, cin_v, o8_v):
    cols = (c0, c1, c2, c3, c4, c5)
    rows = c0.shape[0]
    nchunk = -(-rows // _CROWS)
    tail = rows - (nchunk - 1) * _CROWS
    wid = lax.axis_index("s") * _NC + lax.axis_index("c")
    steps = -(-nchunk // _NW)
    iota = lax.iota(jnp.int32, _L)

    @pl.loop(0, steps)
    def _step(k):
        cid = k * _NW + wid
        rb = cid * _CROWS

        @pl.when(cid < nchunk - 1)
        def _full():
            for ch in range(6):
                pltpu.sync_copy(cols[ch].at[pl.ds(rb, _CROWS)], cin_v.at[ch])
            for vg in range(_CROWS // _L):
                rvec = iota + vg * _L
                for ch in range(6):
                    v = cin_v[ch, pl.ds(vg * _L, _L)]
                    plsc.store_scatter(o8_v, [rvec, jnp.full((_L,), ch, jnp.int32)], v)
            pltpu.sync_copy(o8_v, e8_hbm.at[pl.ds(rb, _CROWS), :])

        @pl.when(cid == nchunk - 1)
        def _tail():
            for ch in range(6):
                pltpu.sync_copy(cols[ch].at[pl.ds(rb, tail)],
                                cin_v.at[ch, pl.ds(0, tail)])
            for vg in range(-(-tail // _L)):
                rvec = iota + vg * _L
                full = (vg + 1) * _L <= tail
                mask = None if full else rvec < tail
                for ch in range(6):
                    v = cin_v[ch, pl.ds(vg * _L, _L)]
                    plsc.store_scatter(o8_v, [rvec, jnp.full((_L,), ch, jnp.int32)],
                                       v, mask=mask)
            pltpu.sync_copy(o8_v.at[pl.ds(0, tail), :],
                            e8_hbm.at[pl.ds(rb, tail), :])


def _body(x_hbm, emb_hbm, o_hbm, pts_v, idx_v, w_v, rows_v, out_v, gsem):
    chunk = x_hbm.shape[1] // _NW
    ngroups = chunk // _L
    wid = lax.axis_index("s") * _NC + lax.axis_index("c")
    base = wid * chunk
    pltpu.sync_copy(x_hbm.at[:, pl.ds(base, chunk)], pts_v)
    iota = lax.iota(jnp.int32, _L)

    @pl.loop(0, ngroups)
    def _group(g):
        gb = g * _L
        xs = pts_v[0, pl.ds(gb, _L)]
        ys = pts_v[1, pl.ds(gb, _L)]
        zs = pts_v[2, pl.ds(gb, _L)]
        x0 = (xs + 1.0) * 0.5
        y0 = (ys + 1.0) * 0.5
        z0 = (zs + 1.0) * 0.5

        for l in range(_NUM_LEVELS):
            scale = float(2.0 ** l * _BASE_RES - 1.0)
            px = x0 * scale + 0.5
            py = y0 * scale + 0.5
            pz = z0 * scale + 0.5
            ix = px.astype(jnp.int32)
            iy = py.astype(jnp.int32)
            iz = pz.astype(jnp.int32)
            fx = px - ix.astype(jnp.float32)
            fy = py - iy.astype(jnp.float32)
            fz = pz - iz.astype(jnp.float32)
            if l < _DENSE_LEVELS:
                mult = 2 ** l * _BASE_RES + 1
                aa = (ix + _OFFS[l], ix + (_OFFS[l] + 1))
                b0 = iy * mult
                c0 = iz * (mult * mult)
                bb = (b0, b0 + mult)
                cc = (c0, c0 + mult * mult)
            else:
                aa = (ix, ix + 1)
                b0 = iy * _P1
                c0 = iz * _P2
                bb = (b0, b0 + _P1)
                cc = (c0, c0 + _P2)
            wx = (1.0 - fx, fx)
            wy = (1.0 - fy, fy)
            wz = (1.0 - fz, fz)
            wxy = [wx[cx] * wy[cy] for cy in range(2) for cx in range(2)]
            for c in range(_NCORN):
                cx, cy, cz = c & 1, (c >> 1) & 1, (c >> 2) & 1
                if l < _DENSE_LEVELS:
                    idx = aa[cx] + bb[cy] + cc[cz]
                else:
                    idx = ((aa[cx] ^ bb[cy] ^ cc[cz]) & _HASH_MASK) + _OFFS[l]
                idx_v[l, pl.ds(c * _L, _L)] = idx
                w_v[pl.ds((l * _NCORN + c) * _L, _L)] = wxy[cy * 2 + cx] * wz[cz]

        cps = [
            pltpu.async_copy(
                emb_hbm.at[idx_v.at[l]],
                rows_v.at[pl.ds(l * _IDXROW, _IDXROW), :],
                gsem,
            )
            for l in range(_NUM_LEVELS)
        ]
        for cp in cps:
            cp.wait()

        chvecs = [jnp.full((_L,), ch, jnp.int32) for ch in range(6)]
        for l in range(_NUM_LEVELS):
            accs = [jnp.zeros((_L,), jnp.float32) for _ in range(6)]
            for c in range(_NCORN):
                w = w_v[pl.ds((l * _NCORN + c) * _L, _L)]
                rvec = iota + (l * _IDXROW + c * _L)
                for ch in range(6):
                    v = plsc.load_gather(rows_v, [rvec, chvecs[ch]])
                    accs[ch] = accs[ch] + w * v
            for ch in range(6):
                j = (2 * l + (ch % 2)) * 3 + ch // 2
                plsc.store_scatter(out_v, [iota, jnp.full((_L,), j, jnp.int32)],
                                   accs[ch])

        pltpu.sync_copy(out_v, o_hbm.at[pl.ds(base + gb, _L), :])


def _sc_params():
    cp = pltpu.CompilerParams()
    if "needs_layout_passes" in pltpu.CompilerParams.__dataclass_fields__:
        cp = dataclasses.replace(cp, needs_layout_passes=False)
    if "use_tc_tiling_on_sc" in pltpu.CompilerParams.__dataclass_fields__:
        cp = dataclasses.replace(cp, use_tc_tiling_on_sc=False)
    return cp


@jax.jit
def _encode(inputs, emb_x, emb_y, emb_z):
    b = inputs.shape[0]
    x_t = inputs.T
    cols = _split_columns(emb_x, emb_y, emb_z)
    rows = emb_x.shape[0]
    f = _NUM_LEVELS * _LEVEL_DIM * _INPUT_DIM
    mesh = plsc.VectorSubcoreMesh(core_axis_name="c", subcore_axis_name="s")
    interleave = pl.kernel(
        _interleave_body,
        out_type=jax.ShapeDtypeStruct((rows, 8), jnp.float32),
        compiler_params=_sc_params(),
        mesh=mesh,
        scratch_types=[
            pltpu.VMEM((6, _CROWS), jnp.float32),
            pltpu.VMEM((_CROWS, 8), jnp.float32),
        ],
    )
    emb8 = interleave(*cols)
    run = pl.kernel(
        _body,
        out_type=jax.ShapeDtypeStruct((b, f), jnp.float32),
        compiler_params=_sc_params(),
        mesh=mesh,
        scratch_types=[
            pltpu.VMEM((_INPUT_DIM, b // _NW), jnp.float32),
            pltpu.VMEM((_NUM_LEVELS, _IDXROW), jnp.int32),
            pltpu.VMEM((_NUM_LEVELS * _IDXROW,), jnp.float32),
            pltpu.VMEM((_NUM_LEVELS * _IDXROW, 8), jnp.float32),
            pltpu.VMEM((_L, f), jnp.float32),
            pltpu.SemaphoreType.DMA,
        ],
    )
    return run(x_t, emb8)


def kernel(inputs, emb_x, emb_y, emb_z):
    b = inputs.shape[0]
    out = _encode(inputs, emb_x, emb_y, emb_z)
    return out.reshape(b, _NUM_LEVELS * _LEVEL_DIM, _INPUT_DIM)


# speedup vs baseline: 7.1567x; 1.3737x over previous
import dataclasses

import jax
import jax.numpy as jnp
from jax import lax
from jax.experimental import pallas as pl
from jax.experimental.pallas import tpu as pltpu
from jax.experimental.pallas import tpu_sc as plsc

_INPUT_DIM = 3
_NUM_LEVELS = 16
_LEVEL_DIM = 2
_BASE_RES = 16
_HASH_MASK = (1 << 19) - 1
_P1 = 2654435761 - (1 << 32)
_P2 = 805459861

_OFFS = [0, 4920, 40864, 315496, 839784, 1364072, 1888360, 2412648,
         2936936, 3461224, 3985512, 4509800, 5034088, 5558376, 6082664,
         6606952, 7131240]
_TOTAL_ROWS = _OFFS[-1]
_DENSE_LEVELS = 3

_NC, _NS, _L = 2, 16, 16
_NW = _NC * _NS
_NCORN = 8
_IDXROW = _NCORN * _L


_BK = 32768


def _tc_cols_body(xt, yt, zt, o0, o1, o2, o3, o4, o5):
    outs = (o0, o1, o2, o3, o4, o5)
    for i, t in enumerate((xt, yt, zt)):
        outs[2 * i][...] = t[0, :]
        outs[2 * i + 1][...] = t[1, :]


def _split_columns(emb_x, emb_y, emb_z):
    rows = emb_x.shape[0]
    grid = (-(-rows // _BK),)
    in_spec = pl.BlockSpec((2, _BK), lambda j: (0, j))
    out_spec = pl.BlockSpec((_BK,), lambda j: (j,))
    return pl.pallas_call(
        _tc_cols_body,
        grid=grid,
        in_specs=[in_spec] * 3,
        out_specs=[out_spec] * 6,
        out_shape=[jax.ShapeDtypeStruct((rows,), jnp.float32)] * 6,
    )(emb_x.T, emb_y.T, emb_z.T)


_CROWS = 2048


def _interleave_body(c0, c1, c2, c3, c4, c5, e8_hbm, cin_v, o8_v):
    cols = (c0, c1, c2, c3, c4, c5)
    rows = c0.shape[0]
    nchunk = -(-rows // _CROWS)
    tail = rows - (nchunk - 1) * _CROWS
    wid = lax.axis_index("s") * _NC + lax.axis_index("c")
    steps = -(-nchunk // _NW)
    iota = lax.iota(jnp.int32, _L)

    @pl.loop(0, steps)
    def _step(k):
        cid = k * _NW + wid
        rb = cid * _CROWS

        @pl.when(cid < nchunk - 1)
        def _full():
            for ch in range(6):
                pltpu.sync_copy(cols[ch].at[pl.ds(rb, _CROWS)], cin_v.at[ch])
            for vg in range(_CROWS // _L):
                rvec = iota + vg * _L
                for ch in range(6):
                    v = cin_v[ch, pl.ds(vg * _L, _L)]
                    plsc.store_scatter(o8_v, [rvec, jnp.full((_L,), ch, jnp.int32)], v)
            pltpu.sync_copy(o8_v, e8_hbm.at[pl.ds(rb, _CROWS), :])

        @pl.when(cid == nchunk - 1)
        def _tail():
            for ch in range(6):
                pltpu.sync_copy(cols[ch].at[pl.ds(rb, tail)],
                                cin_v.at[ch, pl.ds(0, tail)])
            for vg in range(-(-tail // _L)):
                rvec = iota + vg * _L
                full = (vg + 1) * _L <= tail
                mask = None if full else rvec < tail
                for ch in range(6):
                    v = cin_v[ch, pl.ds(vg * _L, _L)]
                    plsc.store_scatter(o8_v, [rvec, jnp.full((_L,), ch, jnp.int32)],
                                       v, mask=mask)
            pltpu.sync_copy(o8_v.at[pl.ds(0, tail), :],
                            e8_hbm.at[pl.ds(rb, tail), :])


_GROWS = _NUM_LEVELS * _IDXROW


def _body(x_hbm, emb_hbm, o_hbm, pts_v, idx_v, w_v, rows_v, out_v, gsem):
    chunk = x_hbm.shape[1] // _NW
    ngroups = chunk // _L
    wid = lax.axis_index("s") * _NC + lax.axis_index("c")
    base = wid * chunk
    pltpu.sync_copy(x_hbm.at[:, pl.ds(base, chunk)], pts_v)
    iota = lax.iota(jnp.int32, _L)
    chvecs = [jnp.full((_L,), ch, jnp.int32) for ch in range(6)]

    def level_store_fire(p, l, aa, bb, cc, wx, wy, wz, dense, off):
        wxy = [wx[cx] * wy[cy] for cy in range(2) for cx in range(2)]
        for c in range(_NCORN):
            cx, cy, cz = c & 1, (c >> 1) & 1, (c >> 2) & 1
            if dense:
                idx = aa[cx] + bb[cy] + cc[cz]
            else:
                idx = ((aa[cx] ^ bb[cy] ^ cc[cz]) & _HASH_MASK) + off
            idx_v[p * _NUM_LEVELS + l, pl.ds(c * _L, _L)] = idx
            w_v[pl.ds((p * _NUM_LEVELS + l) * _IDXROW + c * _L, _L)] = \
                wxy[cy * 2 + cx] * wz[cz]
        pltpu.async_copy(
            emb_hbm.at[idx_v.at[p * _NUM_LEVELS + l]],
            rows_v.at[pl.ds((p * _NUM_LEVELS + l) * _IDXROW, _IDXROW), :],
            gsem.at[p],
        )

    def phase_a(g, p):
        gb = g * _L
        xs = pts_v[0, pl.ds(gb, _L)]
        ys = pts_v[1, pl.ds(gb, _L)]
        zs = pts_v[2, pl.ds(gb, _L)]
        x0 = (xs + 1.0) * 0.5
        y0 = (ys + 1.0) * 0.5
        z0 = (zs + 1.0) * 0.5
        for l in range(_DENSE_LEVELS):
            scale = float(2.0 ** l * _BASE_RES - 1.0)
            px, py, pz = x0 * scale + 0.5, y0 * scale + 0.5, z0 * scale + 0.5
            ix, iy, iz = (px.astype(jnp.int32), py.astype(jnp.int32),
                          pz.astype(jnp.int32))
            fx = px - ix.astype(jnp.float32)
            fy = py - iy.astype(jnp.float32)
            fz = pz - iz.astype(jnp.float32)
            mult = 2 ** l * _BASE_RES + 1
            aa = (ix + _OFFS[l], ix + (_OFFS[l] + 1))
            b0 = iy * mult
            c0 = iz * (mult * mult)
            level_store_fire(p, l, aa, (b0, b0 + mult),
                             (c0, c0 + mult * mult),
                             (1.0 - fx, fx), (1.0 - fy, fy), (1.0 - fz, fz),
                             True, 0)

        @pl.loop(_DENSE_LEVELS, _NUM_LEVELS)
        def _hashed(l):
            scale = lax.shift_left(jnp.int32(_BASE_RES), l).astype(
                jnp.float32) - 1.0
            off = _OFFS[_DENSE_LEVELS] + (l - _DENSE_LEVELS) * (1 << 19)
            px, py, pz = x0 * scale + 0.5, y0 * scale + 0.5, z0 * scale + 0.5
            ix, iy, iz = (px.astype(jnp.int32), py.astype(jnp.int32),
                          pz.astype(jnp.int32))
            fx = px - ix.astype(jnp.float32)
            fy = py - iy.astype(jnp.float32)
            fz = pz - iz.astype(jnp.float32)
            b0 = iy * _P1
            c0 = iz * _P2
            level_store_fire(p, l, (ix, ix + 1), (b0, b0 + _P1),
                             (c0, c0 + _P2),
                             (1.0 - fx, fx), (1.0 - fy, fy), (1.0 - fz, fz),
                             False, off)

    def phase_c(g, p):
        pltpu.make_async_copy(
            emb_hbm.at[pl.ds(0, _GROWS), :],
            rows_v.at[pl.ds(p * _GROWS, _GROWS), :],
            gsem.at[p],
        ).wait()

        @pl.loop(0, _NUM_LEVELS)
        def _lvl(l):
            rbase = (p * _NUM_LEVELS + l) * _IDXROW
            accs = [jnp.zeros((_L,), jnp.float32) for _ in range(6)]
            for c in range(_NCORN):
                w = w_v[pl.ds(rbase + c * _L, _L)]
                rvec = iota + (rbase + c * _L)
                for ch in range(6):
                    v = plsc.load_gather(rows_v, [rvec, chvecs[ch]])
                    accs[ch] = accs[ch] + w * v
            for ch in range(6):
                jvec = (2 * l + (ch % 2)) * 3 + ch // 2 + jnp.zeros(
                    (_L,), jnp.int32)
                plsc.store_scatter(out_v, [iota, jvec], accs[ch])

        pltpu.sync_copy(out_v, o_hbm.at[pl.ds(base + g * _L, _L), :])

    @pl.loop(0, ngroups)
    def _group(g):
        p = lax.rem(g, 2)
        phase_a(g, p)

        @pl.when(g > 0)
        def _():
            phase_c(g - 1, 1 - p)

    phase_c(ngroups - 1, lax.rem(ngroups - 1, 2))


def _sc_params():
    cp = pltpu.CompilerParams()
    if "needs_layout_passes" in pltpu.CompilerParams.__dataclass_fields__:
        cp = dataclasses.replace(cp, needs_layout_passes=False)
    if "use_tc_tiling_on_sc" in pltpu.CompilerParams.__dataclass_fields__:
        cp = dataclasses.replace(cp, use_tc_tiling_on_sc=False)
    return cp


@jax.jit
def _encode(inputs, emb_x, emb_y, emb_z):
    b = inputs.shape[0]
    x_t = inputs.T
    cols = _split_columns(emb_x, emb_y, emb_z)
    rows = emb_x.shape[0]
    f = _NUM_LEVELS * _LEVEL_DIM * _INPUT_DIM
    mesh = plsc.VectorSubcoreMesh(core_axis_name="c", subcore_axis_name="s")
    interleave = pl.kernel(
        _interleave_body,
        out_type=jax.ShapeDtypeStruct((rows, 8), jnp.float32),
        compiler_params=_sc_params(),
        mesh=mesh,
        scratch_types=[
            pltpu.VMEM((6, _CROWS), jnp.float32),
            pltpu.VMEM((_CROWS, 8), jnp.float32),
        ],
    )
    emb8 = interleave(*cols)
    run = pl.kernel(
        _body,
        out_type=jax.ShapeDtypeStruct((b, f), jnp.float32),
        compiler_params=_sc_params(),
        mesh=mesh,
        scratch_types=[
            pltpu.VMEM((_INPUT_DIM, b // _NW), jnp.float32),
            pltpu.VMEM((2 * _NUM_LEVELS, _IDXROW), jnp.int32),
            pltpu.VMEM((2 * _GROWS,), jnp.float32),
            pltpu.VMEM((2 * _GROWS, 8), jnp.float32),
            pltpu.VMEM((_L, f), jnp.float32),
            pltpu.SemaphoreType.DMA((2,)),
        ],
    )
    return run(x_t, emb8)


def kernel(inputs, emb_x, emb_y, emb_z):
    b = inputs.shape[0]
    out = _encode(inputs, emb_x, emb_y, emb_z)
    return out.reshape(b, _NUM_LEVELS * _LEVEL_DIM, _INPUT_DIM)


# speedup vs baseline: 10.4157x; 1.4554x over previous
import dataclasses

import jax
import jax.numpy as jnp
from jax import lax
from jax.experimental import pallas as pl
from jax.experimental.pallas import tpu as pltpu
from jax.experimental.pallas import tpu_sc as plsc

_INPUT_DIM = 3
_NUM_LEVELS = 16
_LEVEL_DIM = 2
_BASE_RES = 16
_HASH_MASK = (1 << 19) - 1
_P1 = 2654435761 - (1 << 32)
_P2 = 805459861

_OFFS = [0, 4920, 40864, 315496, 839784, 1364072, 1888360, 2412648,
         2936936, 3461224, 3985512, 4509800, 5034088, 5558376, 6082664,
         6606952, 7131240]
_TOTAL_ROWS = _OFFS[-1]
_DENSE_LEVELS = 3

_NC, _NS, _L = 2, 16, 16
_NW = _NC * _NS
_NCORN = 8
_IDXROW = _NCORN * _L


_BK = 32768


def _tc_cols_body(xt, yt, zt, o0, o1, o2, o3, o4, o5):
    outs = (o0, o1, o2, o3, o4, o5)
    for i, t in enumerate((xt, yt, zt)):
        outs[2 * i][...] = t[0, :]
        outs[2 * i + 1][...] = t[1, :]


def _split_columns(emb_x, emb_y, emb_z):
    rows = emb_x.shape[0]
    grid = (-(-rows // _BK),)
    in_spec = pl.BlockSpec((2, _BK), lambda j: (0, j))
    out_spec = pl.BlockSpec((_BK,), lambda j: (j,))
    return pl.pallas_call(
        _tc_cols_body,
        grid=grid,
        in_specs=[in_spec] * 3,
        out_specs=[out_spec] * 6,
        out_shape=[jax.ShapeDtypeStruct((rows,), jnp.float32)] * 6,
    )(emb_x.T, emb_y.T, emb_z.T)


_CROWS = 2048


def _interleave_body(c0, c1, c2, c3, c4, c5, e8_hbm, cin_v, o8_v, isem, osem):
    cols = (c0, c1, c2, c3, c4, c5)
    rows = c0.shape[0]
    nfull = rows // _CROWS
    tail = rows - nfull * _CROWS
    twid = nfull % _NW
    wid = lax.axis_index("s") * _NC + lax.axis_index("c")
    steps = -(-nfull // _NW)
    iota = lax.iota(jnp.int32, _L)
    chvecs = [jnp.full((_L,), ch, jnp.int32) for ch in range(6)]

    def fire_in(k, p):
        cid = k * _NW + wid

        @pl.when(cid < nfull)
        def _():
            for ch in range(6):
                pltpu.async_copy(cols[ch].at[pl.ds(cid * _CROWS, _CROWS)],
                                 cin_v.at[p, ch], isem.at[p])

    fire_in(0, 0)

    @pl.loop(0, steps)
    def _step(k):
        p = lax.rem(k, 2)
        fire_in(k + 1, 1 - p)
        cid = k * _NW + wid

        @pl.when((k >= 2) & ((k - 2) * _NW + wid < nfull))
        def _drain_prev():
            pltpu.make_async_copy(e8_hbm.at[pl.ds(0, _CROWS), :],
                                  o8_v.at[pl.ds(0, _CROWS), :],
                                  osem.at[p]).wait()

        @pl.when(cid < nfull)
        def _():
            for ch in range(6):
                pltpu.make_async_copy(cols[ch].at[pl.ds(0, _CROWS)],
                                      cin_v.at[p, ch], isem.at[p]).wait()

            obase = p * _CROWS
            for vg in range(_CROWS // _L):
                rvec = iota + (obase + vg * _L)
                vs = [cin_v[p, ch, pl.ds(vg * _L, _L)] for ch in range(6)]
                for ch in range(6):
                    plsc.store_scatter(o8_v, [rvec, chvecs[ch]], vs[ch])
            pltpu.async_copy(o8_v.at[pl.ds(obase, _CROWS), :],
                             e8_hbm.at[pl.ds(cid * _CROWS, _CROWS), :],
                             osem.at[p])

    for kf in (steps - 2, steps - 1):
        if kf >= 0:
            @pl.when(kf * _NW + wid < nfull)
            def _drain(kf=kf):
                pltpu.make_async_copy(e8_hbm.at[pl.ds(0, _CROWS), :],
                                      o8_v.at[pl.ds(0, _CROWS), :],
                                      osem.at[kf % 2]).wait()

    if tail:
        @pl.when(wid == twid)
        def _tail():
            rb = nfull * _CROWS
            for ch in range(6):
                pltpu.sync_copy(cols[ch].at[pl.ds(rb, tail)],
                                cin_v.at[0, ch, pl.ds(0, tail)])
            for vg in range(-(-tail // _L)):
                rvec = iota + vg * _L
                mask = None if (vg + 1) * _L <= tail else rvec < tail
                for ch in range(6):
                    v = cin_v[0, ch, pl.ds(vg * _L, _L)]
                    plsc.store_scatter(o8_v, [rvec, chvecs[ch]], v, mask=mask)
            pltpu.sync_copy(o8_v.at[pl.ds(0, tail), :],
                            e8_hbm.at[pl.ds(rb, tail), :])


_GROWS = _NUM_LEVELS * _IDXROW


def _body(x_hbm, emb_hbm, o_hbm, pts_v, idx_v, w_v, rows_v, out_v, gsem):
    chunk = x_hbm.shape[1] // _NW
    ngroups = chunk // _L
    wid = lax.axis_index("s") * _NC + lax.axis_index("c")
    base = wid * chunk
    pltpu.sync_copy(x_hbm.at[:, pl.ds(base, chunk)], pts_v)
    iota = lax.iota(jnp.int32, _L)
    chvecs = [jnp.full((_L,), ch, jnp.int32) for ch in range(6)]

    def level_store_fire(p, l, aa, bb, cc, wx, wy, wz, dense, off):
        wxy = [wx[cx] * wy[cy] for cy in range(2) for cx in range(2)]
        for c in range(_NCORN):
            cx, cy, cz = c & 1, (c >> 1) & 1, (c >> 2) & 1
            if dense:
                idx = aa[cx] + bb[cy] + cc[cz]
            else:
                idx = ((aa[cx] ^ bb[cy] ^ cc[cz]) & _HASH_MASK) + off
            idx_v[p * _NUM_LEVELS + l, pl.ds(c * _L, _L)] = idx
            w_v[pl.ds((p * _NUM_LEVELS + l) * _IDXROW + c * _L, _L)] = \
                wxy[cy * 2 + cx] * wz[cz]
        pltpu.async_copy(
            emb_hbm.at[idx_v.at[p * _NUM_LEVELS + l]],
            rows_v.at[pl.ds((p * _NUM_LEVELS + l) * _IDXROW, _IDXROW), :],
            gsem.at[p],
        )

    def phase_a(g, p):
        gb = g * _L
        xs = pts_v[0, pl.ds(gb, _L)]
        ys = pts_v[1, pl.ds(gb, _L)]
        zs = pts_v[2, pl.ds(gb, _L)]
        x0 = (xs + 1.0) * 0.5
        y0 = (ys + 1.0) * 0.5
        z0 = (zs + 1.0) * 0.5
        for l in range(_DENSE_LEVELS):
            scale = float(2.0 ** l * _BASE_RES - 1.0)
            px, py, pz = x0 * scale + 0.5, y0 * scale + 0.5, z0 * scale + 0.5
            ix, iy, iz = (px.astype(jnp.int32), py.astype(jnp.int32),
                          pz.astype(jnp.int32))
            fx = px - ix.astype(jnp.float32)
            fy = py - iy.astype(jnp.float32)
            fz = pz - iz.astype(jnp.float32)
            mult = 2 ** l * _BASE_RES + 1
            aa = (ix + _OFFS[l], ix + (_OFFS[l] + 1))
            b0 = iy * mult
            c0 = iz * (mult * mult)
            level_store_fire(p, l, aa, (b0, b0 + mult),
                             (c0, c0 + mult * mult),
                             (1.0 - fx, fx), (1.0 - fy, fy), (1.0 - fz, fz),
                             True, 0)

        @pl.loop(_DENSE_LEVELS, _NUM_LEVELS)
        def _hashed(l):
            scale = lax.shift_left(jnp.int32(_BASE_RES), l).astype(
                jnp.float32) - 1.0
            off = _OFFS[_DENSE_LEVELS] + (l - _DENSE_LEVELS) * (1 << 19)
            px, py, pz = x0 * scale + 0.5, y0 * scale + 0.5, z0 * scale + 0.5
            ix, iy, iz = (px.astype(jnp.int32), py.astype(jnp.int32),
                          pz.astype(jnp.int32))
            fx = px - ix.astype(jnp.float32)
            fy = py - iy.astype(jnp.float32)
            fz = pz - iz.astype(jnp.float32)
            b0 = iy * _P1
            c0 = iz * _P2
            level_store_fire(p, l, (ix, ix + 1), (b0, b0 + _P1),
                             (c0, c0 + _P2),
                             (1.0 - fx, fx), (1.0 - fy, fy), (1.0 - fz, fz),
                             False, off)

    def phase_c(g, p):
        pltpu.make_async_copy(
            emb_hbm.at[pl.ds(0, _GROWS), :],
            rows_v.at[pl.ds(p * _GROWS, _GROWS), :],
            gsem.at[p],
        ).wait()

        @pl.loop(0, _NUM_LEVELS)
        def _lvl(l):
            rbase = (p * _NUM_LEVELS + l) * _IDXROW
            accs = [jnp.zeros((_L,), jnp.float32) for _ in range(6)]
            for c in range(_NCORN):
                w = w_v[pl.ds(rbase + c * _L, _L)]
                rvec = iota + (rbase + c * _L)
                for ch in range(6):
                    v = plsc.load_gather(rows_v, [rvec, chvecs[ch]])
                    accs[ch] = accs[ch] + w * v
            for ch in range(6):
                jvec = (2 * l + (ch % 2)) * 3 + ch // 2 + jnp.zeros(
                    (_L,), jnp.int32)
                plsc.store_scatter(out_v, [iota, jvec], accs[ch])

        pltpu.sync_copy(out_v, o_hbm.at[pl.ds(base + g * _L, _L), :])

    @pl.loop(0, ngroups)
    def _group(g):
        p = lax.rem(g, 2)
        phase_a(g, p)

        @pl.when(g > 0)
        def _():
            phase_c(g - 1, 1 - p)

    phase_c(ngroups - 1, lax.rem(ngroups - 1, 2))


def _sc_params():
    cp = pltpu.CompilerParams()
    if "needs_layout_passes" in pltpu.CompilerParams.__dataclass_fields__:
        cp = dataclasses.replace(cp, needs_layout_passes=False)
    if "use_tc_tiling_on_sc" in pltpu.CompilerParams.__dataclass_fields__:
        cp = dataclasses.replace(cp, use_tc_tiling_on_sc=False)
    return cp


@jax.jit
def _encode(inputs, emb_x, emb_y, emb_z):
    b = inputs.shape[0]
    x_t = inputs.T
    cols = _split_columns(emb_x, emb_y, emb_z)
    rows = emb_x.shape[0]
    f = _NUM_LEVELS * _LEVEL_DIM * _INPUT_DIM
    mesh = plsc.VectorSubcoreMesh(core_axis_name="c", subcore_axis_name="s")
    interleave = pl.kernel(
        _interleave_body,
        out_type=jax.ShapeDtypeStruct((rows, 8), jnp.float32),
        compiler_params=_sc_params(),
        mesh=mesh,
        scratch_types=[
            pltpu.VMEM((2, 6, _CROWS), jnp.float32),
            pltpu.VMEM((2 * _CROWS, 8), jnp.float32),
            pltpu.SemaphoreType.DMA((2,)),
            pltpu.SemaphoreType.DMA((2,)),
        ],
    )
    emb8 = interleave(*cols)
    run = pl.kernel(
        _body,
        out_type=jax.ShapeDtypeStruct((b, f), jnp.float32),
        compiler_params=_sc_params(),
        mesh=mesh,
        scratch_types=[
            pltpu.VMEM((_INPUT_DIM, b // _NW), jnp.float32),
            pltpu.VMEM((2 * _NUM_LEVELS, _IDXROW), jnp.int32),
            pltpu.VMEM((2 * _GROWS,), jnp.float32),
            pltpu.VMEM((2 * _GROWS, 8), jnp.float32),
            pltpu.VMEM((_L, f), jnp.float32),
            pltpu.SemaphoreType.DMA((2,)),
        ],
    )
    return run(x_t, emb8)


def kernel(inputs, emb_x, emb_y, emb_z):
    b = inputs.shape[0]
    out = _encode(inputs, emb_x, emb_y, emb_z)
    return out.reshape(b, _NUM_LEVELS * _LEVEL_DIM, _INPUT_DIM)
